# baseline (speedup 1.0000x reference)
import jax
import jax.numpy as jnp
from jax.experimental import pallas as pl
from jax.experimental.pallas import tpu as pltpu

N = 4194304
WIDTH = 2048
HEIGHT = 2048
P_T = 0.01

ROWS = 4096
COLS = 1024
BLOCK_ROWS = 512
GRID = ROWS // BLOCK_ROWS

_KEYS = (
    (0x1E3F1835, 0x6E752082),
    (0x74298876, 0xFC8D8048),
    (0x918CA911, 0xE3AB1C6B),
)

_ROTS = (13, 15, 26, 6, 17, 29, 16, 24)
_M32 = 0xFFFFFFFF


def _threefry_bits(lo, k0, k1):
    ks2 = (k0 ^ k1 ^ 0x1BD11BDA) & _M32
    ks = (k0, k1, ks2)
    x0 = jnp.full(lo.shape, jnp.uint32(k0))
    x1 = lo + jnp.uint32(k1)
    for g in range(5):
        rots = _ROTS[0:4] if g % 2 == 0 else _ROTS[4:8]
        for r in rots:
            x0 = x0 + x1
            x1 = ((x1 << jnp.uint32(r)) | (x1 >> jnp.uint32(32 - r))) ^ x0
        x0 = x0 + jnp.uint32(ks[(g + 1) % 3])
        x1 = x1 + jnp.uint32((ks[(g + 2) % 3] + g + 1) & _M32)
    return x0 ^ x1


def _uniform(bits):
    f = jax.lax.bitcast_convert_type(
        (bits >> jnp.uint32(9)) | jnp.uint32(0x3F800000), jnp.float32)
    return f - jnp.float32(1.0)


def _body(x_ref, y_ref, t_ref, xo_ref, yo_ref, to_ref):
    pid = pl.program_id(0)
    shape = (BLOCK_ROWS, COLS)
    row = jax.lax.broadcasted_iota(jnp.uint32, shape, 0)
    col = jax.lax.broadcasted_iota(jnp.uint32, shape, 1)
    lin = pid.astype(jnp.uint32) * jnp.uint32(BLOCK_ROWS * COLS) \
        + row * jnp.uint32(COLS) + col

    two = jnp.float32(2.0)
    pi_ish = jnp.float32(3.141592)
    theta_rand = _uniform(_threefry_bits(lin, *_KEYS[0])) * two * pi_ish
    prob = _uniform(_threefry_bits(lin, *_KEYS[1]))
    theta_rand2 = _uniform(_threefry_bits(lin, *_KEYS[2])) * two * pi_ish

    theta = jnp.where(prob <= jnp.float32(P_T), theta_rand, t_ref[...])
    x = x_ref[...] + jnp.cos(theta)
    y = y_ref[...] + jnp.sin(theta)

    zeros = jnp.float32(0.0)
    ones = jnp.float32(1.0)
    x_lim = jnp.float32(WIDTH - 1)
    y_lim = jnp.float32(HEIGHT - 1)
    x_boxed = jnp.maximum(zeros, jnp.minimum(x, x_lim))
    y_boxed = jnp.maximum(zeros, jnp.minimum(y, y_lim))
    x_hi = x >= jnp.float32(WIDTH)
    x_lo = x <= zeros
    y_hi = y >= jnp.float32(HEIGHT)
    y_lo = y <= zeros
    x_clip = jnp.where(x_lo, x_boxed, jnp.where(x_hi, x_boxed, x))
    y_clip = jnp.where(y_lo, y_boxed, jnp.where(y_hi, y_boxed, y))
    tc = (jnp.where(x_hi, ones, zeros) + jnp.where(x_lo, ones, zeros)
          + jnp.where(y_hi, ones, zeros) + jnp.where(y_lo, ones, zeros))
    theta_clip = tc * theta_rand2 + jnp.abs(tc - ones) * theta

    xo_ref[...] = x_clip
    yo_ref[...] = y_clip
    to_ref[...] = theta_clip


def kernel(x, y, theta, frame):
    del frame
    xr = x.reshape(ROWS, COLS)
    yr = y.reshape(ROWS, COLS)
    tr = theta.reshape(ROWS, COLS)
    spec = pl.BlockSpec((BLOCK_ROWS, COLS), lambda i: (i, 0))
    out = pl.pallas_call(
        _body,
        grid=(GRID,),
        in_specs=[spec, spec, spec],
        out_specs=[spec, spec, spec],
        out_shape=[jax.ShapeDtypeStruct((ROWS, COLS), jnp.float32)] * 3,
        compiler_params=pltpu.CompilerParams(
            dimension_semantics=("parallel",)),
    )(xr, yr, tr)
    return (out[0].reshape(N), out[1].reshape(N), out[2].reshape(N))


# speedup vs baseline: 1.1995x; 1.1995x over previous
import jax
import jax.numpy as jnp
from jax.experimental import pallas as pl
from jax.experimental.pallas import tpu as pltpu

N = 4194304
WIDTH = 2048
HEIGHT = 2048
P_T = 0.01

ROWS = 4096
COLS = 1024
BLOCK_ROWS = 512
GRID = ROWS // BLOCK_ROWS

_KEYS = (
    (0x1E3F1835, 0x6E752082),
    (0x74298876, 0xFC8D8048),
    (0x918CA911, 0xE3AB1C6B),
)

_ROTS = (13, 15, 26, 6, 17, 29, 16, 24)
_M32 = 0xFFFFFFFF


def _threefry_bits(lo, k0, k1):
    ks2 = (k0 ^ k1 ^ 0x1BD11BDA) & _M32
    ks = (k0, k1, ks2)
    x0 = jnp.full(lo.shape, jnp.uint32(k0))
    x1 = lo + jnp.uint32(k1)
    for g in range(5):
        rots = _ROTS[0:4] if g % 2 == 0 else _ROTS[4:8]
        for r in rots:
            x0 = x0 + x1
            x1 = ((x1 << jnp.uint32(r)) | (x1 >> jnp.uint32(32 - r))) ^ x0
        x0 = x0 + jnp.uint32(ks[(g + 1) % 3])
        x1 = x1 + jnp.uint32((ks[(g + 2) % 3] + g + 1) & _M32)
    return x0 ^ x1


_COS_COEFS = (-0.9999999922905627, 0.49999991772620556, -0.041666524364311605,
              0.001388797040930088, -2.477342416861343e-05,
              2.7113373005064955e-07, -1.7369132868694975e-09)
_SIN_COEFS = (-0.9999999994768278, 0.16666666108558614, -0.008333323685061518,
              0.00019840647541474363, -2.7538258026080384e-06,
              2.4752169004531126e-08, -1.3697464223144177e-10)
_PI = 3.14159265358979


def _cos_sin(theta):
    u = theta - jnp.float32(_PI)
    w = u * u
    c = jnp.float32(_COS_COEFS[6])
    for k in range(5, -1, -1):
        c = c * w + jnp.float32(_COS_COEFS[k])
    s = jnp.float32(_SIN_COEFS[6])
    for k in range(5, -1, -1):
        s = s * w + jnp.float32(_SIN_COEFS[k])
    return c, s * u


def _uniform(bits):
    f = jax.lax.bitcast_convert_type(
        (bits >> jnp.uint32(9)) | jnp.uint32(0x3F800000), jnp.float32)
    return f - jnp.float32(1.0)


def _body(x_ref, y_ref, t_ref, xo_ref, yo_ref, to_ref):
    pid = pl.program_id(0)
    shape = (BLOCK_ROWS, COLS)
    row = jax.lax.broadcasted_iota(jnp.uint32, shape, 0)
    col = jax.lax.broadcasted_iota(jnp.uint32, shape, 1)
    lin = pid.astype(jnp.uint32) * jnp.uint32(BLOCK_ROWS * COLS) \
        + row * jnp.uint32(COLS) + col

    two = jnp.float32(2.0)
    pi_ish = jnp.float32(3.141592)
    theta_rand = _uniform(_threefry_bits(lin, *_KEYS[0])) * two * pi_ish
    prob = _uniform(_threefry_bits(lin, *_KEYS[1]))
    theta_rand2 = _uniform(_threefry_bits(lin, *_KEYS[2])) * two * pi_ish

    theta = jnp.where(prob <= jnp.float32(P_T), theta_rand, t_ref[...])
    cos_t, sin_t = _cos_sin(theta)
    x = x_ref[...] + cos_t
    y = y_ref[...] + sin_t

    zeros = jnp.float32(0.0)
    ones = jnp.float32(1.0)
    x_lim = jnp.float32(WIDTH - 1)
    y_lim = jnp.float32(HEIGHT - 1)
    x_boxed = jnp.maximum(zeros, jnp.minimum(x, x_lim))
    y_boxed = jnp.maximum(zeros, jnp.minimum(y, y_lim))
    x_hi = x >= jnp.float32(WIDTH)
    x_lo = x <= zeros
    y_hi = y >= jnp.float32(HEIGHT)
    y_lo = y <= zeros
    x_clip = jnp.where(x_lo, x_boxed, jnp.where(x_hi, x_boxed, x))
    y_clip = jnp.where(y_lo, y_boxed, jnp.where(y_hi, y_boxed, y))
    tc = (jnp.where(x_hi, ones, zeros) + jnp.where(x_lo, ones, zeros)
          + jnp.where(y_hi, ones, zeros) + jnp.where(y_lo, ones, zeros))
    theta_clip = tc * theta_rand2 + jnp.abs(tc - ones) * theta

    xo_ref[...] = x_clip
    yo_ref[...] = y_clip
    to_ref[...] = theta_clip


def kernel(x, y, theta, frame):
    del frame
    xr = x.reshape(ROWS, COLS)
    yr = y.reshape(ROWS, COLS)
    tr = theta.reshape(ROWS, COLS)
    spec = pl.BlockSpec((BLOCK_ROWS, COLS), lambda i: (i, 0))
    out = pl.pallas_call(
        _body,
        grid=(GRID,),
        in_specs=[spec, spec, spec],
        out_specs=[spec, spec, spec],
        out_shape=[jax.ShapeDtypeStruct((ROWS, COLS), jnp.float32)] * 3,
        compiler_params=pltpu.CompilerParams(
            dimension_semantics=("parallel",)),
    )(xr, yr, tr)
    return (out[0].reshape(N), out[1].reshape(N), out[2].reshape(N))


# speedup vs baseline: 1.2124x; 1.0108x over previous
import jax
import jax.numpy as jnp
from jax.experimental import pallas as pl
from jax.experimental.pallas import tpu as pltpu

N = 4194304
WIDTH = 2048
HEIGHT = 2048
P_T = 0.01

ROWS = 4096
COLS = 1024
BLOCK_ROWS = 512
GRID = ROWS // BLOCK_ROWS

_KEYS = (
    (0x1E3F1835, 0x6E752082),
    (0x74298876, 0xFC8D8048),
    (0x918CA911, 0xE3AB1C6B),
)

_ROTS = (13, 15, 26, 6, 17, 29, 16, 24)
_M32 = 0xFFFFFFFF


def _threefry_bits(lo, k0, k1):
    ks2 = (k0 ^ k1 ^ 0x1BD11BDA) & _M32
    ks = (k0, k1, ks2)
    x0 = jnp.full(lo.shape, jnp.uint32(k0))
    x1 = lo + jnp.uint32(k1)
    for g in range(5):
        rots = _ROTS[0:4] if g % 2 == 0 else _ROTS[4:8]
        for r in rots:
            x0 = x0 + x1
            x1 = ((x1 << jnp.uint32(r)) | (x1 >> jnp.uint32(32 - r))) ^ x0
        x0 = x0 + jnp.uint32(ks[(g + 1) % 3])
        x1 = x1 + jnp.uint32((ks[(g + 2) % 3] + g + 1) & _M32)
    return x0 ^ x1


_COS_COEFS = (-0.9999999922905627, 0.49999991772620556, -0.041666524364311605,
              0.001388797040930088, -2.477342416861343e-05,
              2.7113373005064955e-07, -1.7369132868694975e-09)
_SIN_COEFS = (-0.9999999994768278, 0.16666666108558614, -0.008333323685061518,
              0.00019840647541474363, -2.7538258026080384e-06,
              2.4752169004531126e-08, -1.3697464223144177e-10)
_PI = 3.14159265358979


def _cos_sin(theta):
    u = theta - jnp.float32(_PI)
    w = u * u
    c = jnp.float32(_COS_COEFS[6])
    for k in range(5, -1, -1):
        c = c * w + jnp.float32(_COS_COEFS[k])
    s = jnp.float32(_SIN_COEFS[6])
    for k in range(5, -1, -1):
        s = s * w + jnp.float32(_SIN_COEFS[k])
    return c, s * u


_THETA_SCALE = 7.490139099181862524449825286865234375e-07
_PROB_BITS_LE = 42950143


def _bits_to_angle(bits):
    m = (bits >> jnp.uint32(9)).astype(jnp.int32)
    return m.astype(jnp.float32) * jnp.float32(_THETA_SCALE)


def _body(x_ref, y_ref, t_ref, xo_ref, yo_ref, to_ref):
    pid = pl.program_id(0)
    shape = (BLOCK_ROWS, COLS)
    row = jax.lax.broadcasted_iota(jnp.uint32, shape, 0)
    col = jax.lax.broadcasted_iota(jnp.uint32, shape, 1)
    lin = pid.astype(jnp.uint32) * jnp.uint32(BLOCK_ROWS * COLS) \
        + row * jnp.uint32(COLS) + col

    theta_rand = _bits_to_angle(_threefry_bits(lin, *_KEYS[0]))
    prob_bits = _threefry_bits(lin, *_KEYS[1])
    theta_rand2 = _bits_to_angle(_threefry_bits(lin, *_KEYS[2]))

    theta = jnp.where(prob_bits <= jnp.uint32(_PROB_BITS_LE),
                      theta_rand, t_ref[...])
    cos_t, sin_t = _cos_sin(theta)
    x = x_ref[...] + cos_t
    y = y_ref[...] + sin_t

    zeros = jnp.float32(0.0)
    ones = jnp.float32(1.0)
    x_lim = jnp.float32(WIDTH - 1)
    y_lim = jnp.float32(HEIGHT - 1)
    x_boxed = jnp.maximum(zeros, jnp.minimum(x, x_lim))
    y_boxed = jnp.maximum(zeros, jnp.minimum(y, y_lim))
    x_hi = x >= jnp.float32(WIDTH)
    x_lo = x <= zeros
    y_hi = y >= jnp.float32(HEIGHT)
    y_lo = y <= zeros
    x_clip = jnp.where(x_lo, x_boxed, jnp.where(x_hi, x_boxed, x))
    y_clip = jnp.where(y_lo, y_boxed, jnp.where(y_hi, y_boxed, y))
    tc = (jnp.where(x_hi, ones, zeros) + jnp.where(x_lo, ones, zeros)
          + jnp.where(y_hi, ones, zeros) + jnp.where(y_lo, ones, zeros))
    theta_clip = tc * theta_rand2 + jnp.abs(tc - ones) * theta

    xo_ref[...] = x_clip
    yo_ref[...] = y_clip
    to_ref[...] = theta_clip


def kernel(x, y, theta, frame):
    del frame
    xr = x.reshape(ROWS, COLS)
    yr = y.reshape(ROWS, COLS)
    tr = theta.reshape(ROWS, COLS)
    spec = pl.BlockSpec((BLOCK_ROWS, COLS), lambda i: (i, 0))
    out = pl.pallas_call(
        _body,
        grid=(GRID,),
        in_specs=[spec, spec, spec],
        out_specs=[spec, spec, spec],
        out_shape=[jax.ShapeDtypeStruct((ROWS, COLS), jnp.float32)] * 3,
        compiler_params=pltpu.CompilerParams(
            dimension_semantics=("parallel",)),
    )(xr, yr, tr)
    return (out[0].reshape(N), out[1].reshape(N), out[2].reshape(N))
